# Initial kernel scaffold; baseline (speedup 1.0000x reference)
import jax
import jax.numpy as jnp
from jax.experimental import pallas as pl


def kernel(input_word, context_word, W_in, W_ctx):
    raise NotImplementedError("write your pallas kernel here")


# speedup vs baseline: 2.4288x; 2.4288x over previous
import functools

import jax
import jax.numpy as jnp
import numpy as np
from jax import lax
from jax.experimental import pallas as pl
from jax.experimental.pallas import tpu as pltpu
from jax.experimental.pallas import tpu_sc as plsc

_VOCAB = 1000000
_EMB = 64
_NEG = 10
_BATCH = 16384

_NC = 2
_NS = 16
_NW = _NC * _NS
_CH = 128

def _noise_flat():
    nz = jax.random.randint(jax.random.key(42), (_BATCH, _NEG), 0, _VOCAB)
    return nz.astype(jnp.int32).reshape(-1)


def _sc_gather(input_word, context_word, noise_flat, W_in, W_ctx):
    B = _BATCH
    NB = _BATCH * _NEG
    mesh = plsc.VectorSubcoreMesh(core_axis_name="c", subcore_axis_name="s")
    out_types = (
        jax.ShapeDtypeStruct((B, _EMB), jnp.float32),
        jax.ShapeDtypeStruct((B, _EMB), jnp.float32),
        jax.ShapeDtypeStruct((NB, _EMB), jnp.float32),
    )

    @functools.partial(
        pl.kernel,
        mesh=mesh,
        out_type=out_types,
        compiler_params=pltpu.CompilerParams(use_tc_tiling_on_sc=False),
        scratch_types=[
            pltpu.VMEM((_CH,), jnp.int32),
            pltpu.VMEM((_CH, _EMB), jnp.float32),
            pltpu.SemaphoreType.DMA,
        ],
    )
    def k(iw_hbm, cw_hbm, nz_hbm, win_hbm, wctx_hbm,
          oin_hbm, octx_hbm, oneg_hbm, idx_v, rows_v, sem):
        wid = lax.axis_index("s") * _NC + lax.axis_index("c")

        def gather(idx_hbm, table_hbm, out_hbm, per_w):
            base = wid * per_w

            @pl.loop(0, per_w // _CH)
            def _(i):
                off = base + i * _CH
                pltpu.sync_copy(idx_hbm.at[pl.ds(off, _CH)], idx_v)
                pltpu.async_copy(table_hbm.at[idx_v], rows_v, sem).wait()
                pltpu.sync_copy(rows_v, out_hbm.at[pl.ds(off, _CH)])

        gather(iw_hbm, win_hbm, oin_hbm, B // _NW)
        gather(cw_hbm, wctx_hbm, octx_hbm, B // _NW)
        gather(nz_hbm, wctx_hbm, oneg_hbm, NB // _NW)

    return k(input_word, context_word, noise_flat, W_in, W_ctx)


def _tc_loss(emb_in, emb_ctx, emb_neg):
    B = _BATCH
    Bb = 1024
    G = B // Bb

    def body(in_ref, ctx_ref, neg_ref, acc_ref):
        a = in_ref[...]
        c = ctx_ref[...]
        n = neg_ref[...].reshape(Bb, _NEG, _EMB)
        pos = jnp.sum(a * c, axis=1, keepdims=True)
        negs = jnp.sum(n * a[:, None, :], axis=2)

        def logsig(x):
            return jnp.minimum(x, 0.0) - jnp.log1p(jnp.exp(-jnp.abs(x)))

        total = jnp.sum(logsig(pos)) + jnp.sum(logsig(-negs))

        @pl.when(pl.program_id(0) == 0)
        def _():
            acc_ref[...] = jnp.zeros((1, 1), jnp.float32)

        acc_ref[...] += jnp.reshape(total, (1, 1))

    acc = pl.pallas_call(
        body,
        grid=(G,),
        in_specs=[
            pl.BlockSpec((Bb, _EMB), lambda i: (i, 0)),
            pl.BlockSpec((Bb, _EMB), lambda i: (i, 0)),
            pl.BlockSpec((Bb * _NEG, _EMB), lambda i: (i, 0)),
        ],
        out_specs=pl.BlockSpec((1, 1), lambda i: (0, 0)),
        out_shape=jax.ShapeDtypeStruct((1, 1), jnp.float32),
    )(emb_in, emb_ctx, emb_neg)
    return -acc[0, 0] / B


def kernel(input_word, context_word, W_in, W_ctx):
    iw = input_word.astype(jnp.int32)
    cw = context_word.astype(jnp.int32)
    nz = _noise_flat()
    emb_in, emb_ctx, emb_neg = _sc_gather(iw, cw, nz, W_in, W_ctx)
    return _tc_loss(emb_in, emb_ctx, emb_neg)


# speedup vs baseline: 2.5274x; 1.0406x over previous
import functools

import jax
import jax.numpy as jnp
import numpy as np
from jax import lax
from jax.experimental import pallas as pl
from jax.experimental.pallas import tpu as pltpu
from jax.experimental.pallas import tpu_sc as plsc

_VOCAB = 1000000
_EMB = 64
_NEG = 10
_BATCH = 16384

_NC = 2
_NS = 16
_NW = _NC * _NS
_CH = 512

def _noise_flat():
    nz = jax.random.randint(jax.random.key(42), (_BATCH, _NEG), 0, _VOCAB)
    return nz.astype(jnp.int32).reshape(-1)


def _sc_gather(input_word, context_word, noise_flat, W_in, W_ctx):
    B = _BATCH
    NB = _BATCH * _NEG
    mesh = plsc.VectorSubcoreMesh(core_axis_name="c", subcore_axis_name="s")
    out_types = (
        jax.ShapeDtypeStruct((B, _EMB), jnp.float32),
        jax.ShapeDtypeStruct((B, _EMB), jnp.float32),
        jax.ShapeDtypeStruct((NB, _EMB), jnp.float32),
    )

    b_w = B // _NW
    n_w = NB // _NW
    nneg = n_w // _CH

    @functools.partial(
        pl.kernel,
        mesh=mesh,
        out_type=out_types,
        compiler_params=pltpu.CompilerParams(use_tc_tiling_on_sc=False),
        scratch_types=[
            pltpu.VMEM((b_w + b_w + n_w,), jnp.int32),
            pltpu.VMEM((_CH, _EMB), jnp.float32),
            pltpu.VMEM((_CH, _EMB), jnp.float32),
            pltpu.SemaphoreType.DMA,
            pltpu.SemaphoreType.DMA,
        ],
    )
    def k(iw_hbm, cw_hbm, nz_hbm, win_hbm, wctx_hbm,
          oin_hbm, octx_hbm, oneg_hbm, idx_v, rows0, rows1, sem0, sem1):
        wid = lax.axis_index("s") * _NC + lax.axis_index("c")

        pltpu.sync_copy(iw_hbm.at[pl.ds(wid * b_w, b_w)], idx_v.at[pl.ds(0, b_w)])
        pltpu.sync_copy(cw_hbm.at[pl.ds(wid * b_w, b_w)],
                        idx_v.at[pl.ds(b_w, b_w)])
        pltpu.sync_copy(nz_hbm.at[pl.ds(wid * n_w, n_w)],
                        idx_v.at[pl.ds(2 * b_w, n_w)])

        chunks = [(0, win_hbm, oin_hbm, wid * b_w),
                  (b_w, wctx_hbm, octx_hbm, wid * b_w)]
        for j in range(nneg):
            chunks.append((2 * b_w + j * _CH, wctx_hbm, oneg_hbm,
                           wid * n_w + j * _CH))

        bufs, sems = (rows0, rows1), (sem0, sem1)
        copies = [None] * len(chunks)
        for j, (ioff, table, out, ooff) in enumerate(chunks):
            copies[j] = pltpu.async_copy(
                table.at[idx_v.at[pl.ds(ioff, _CH)]], bufs[j % 2], sems[j % 2])
            if j >= 1:
                p_ioff, p_table, p_out, p_ooff = chunks[j - 1]
                copies[j - 1].wait()
                pltpu.sync_copy(bufs[(j - 1) % 2],
                                p_out.at[pl.ds(p_ooff, _CH)])
        copies[-1].wait()
        _, _, l_out, l_ooff = chunks[-1]
        pltpu.sync_copy(bufs[(len(chunks) - 1) % 2],
                        l_out.at[pl.ds(l_ooff, _CH)])

    return k(input_word, context_word, noise_flat, W_in, W_ctx)


def _tc_loss(emb_in, emb_ctx, emb_neg):
    B = _BATCH
    Bb = 1024
    G = B // Bb

    def body(in_ref, ctx_ref, neg_ref, acc_ref):
        a = in_ref[...]
        c = ctx_ref[...]
        n = neg_ref[...].reshape(Bb, _NEG, _EMB)
        pos = jnp.sum(a * c, axis=1, keepdims=True)
        negs = jnp.sum(n * a[:, None, :], axis=2)

        def logsig(x):
            return jnp.minimum(x, 0.0) - jnp.log1p(jnp.exp(-jnp.abs(x)))

        total = jnp.sum(logsig(pos)) + jnp.sum(logsig(-negs))

        @pl.when(pl.program_id(0) == 0)
        def _():
            acc_ref[...] = jnp.zeros((1, 1), jnp.float32)

        acc_ref[...] += jnp.reshape(total, (1, 1))

    acc = pl.pallas_call(
        body,
        grid=(G,),
        in_specs=[
            pl.BlockSpec((Bb, _EMB), lambda i: (i, 0)),
            pl.BlockSpec((Bb, _EMB), lambda i: (i, 0)),
            pl.BlockSpec((Bb * _NEG, _EMB), lambda i: (i, 0)),
        ],
        out_specs=pl.BlockSpec((1, 1), lambda i: (0, 0)),
        out_shape=jax.ShapeDtypeStruct((1, 1), jnp.float32),
    )(emb_in, emb_ctx, emb_neg)
    return -acc[0, 0] / B


def kernel(input_word, context_word, W_in, W_ctx):
    iw = input_word.astype(jnp.int32)
    cw = context_word.astype(jnp.int32)
    nz = _noise_flat()
    emb_in, emb_ctx, emb_neg = _sc_gather(iw, cw, nz, W_in, W_ctx)
    return _tc_loss(emb_in, emb_ctx, emb_neg)


# speedup vs baseline: 2.6831x; 1.0616x over previous
import functools

import jax
import jax.numpy as jnp
from jax import lax
from jax.experimental import pallas as pl
from jax.experimental.pallas import tpu as pltpu
from jax.experimental.pallas import tpu_sc as plsc

_VOCAB = 1000000
_EMB = 64
_PAD = 128
_NEG = 10
_BATCH = 16384

_NC = 2
_NS = 16
_NW = _NC * _NS
_CH = 256


def _noise_flat():
    nz = jax.random.randint(jax.random.key(42), (_BATCH, _NEG), 0, _VOCAB)
    return nz.astype(jnp.int32).reshape(-1)


def _sc_gather(input_word, context_word, noise_flat, W_in, W_ctx):
    B = _BATCH
    NB = _BATCH * _NEG
    mesh = plsc.VectorSubcoreMesh(core_axis_name="c", subcore_axis_name="s")
    out_types = (
        jax.ShapeDtypeStruct((B, _PAD), jnp.float32),
        jax.ShapeDtypeStruct((B, _PAD), jnp.float32),
        jax.ShapeDtypeStruct((NB, _PAD), jnp.float32),
    )

    b_w = B // _NW
    n_w = NB // _NW
    nneg = n_w // _CH

    @functools.partial(
        pl.kernel,
        mesh=mesh,
        out_type=out_types,
        compiler_params=pltpu.CompilerParams(use_tc_tiling_on_sc=False),
        scratch_types=[
            pltpu.VMEM((b_w + b_w + n_w,), jnp.int32),
            pltpu.VMEM((_CH, _EMB), jnp.float32),
            pltpu.VMEM((_CH, _EMB), jnp.float32),
            pltpu.SemaphoreType.DMA,
            pltpu.SemaphoreType.DMA,
        ],
    )
    def k(iw_hbm, cw_hbm, nz_hbm, win_hbm, wctx_hbm,
          oin_hbm, octx_hbm, oneg_hbm, idx_v, rows0, rows1, sem0, sem1):
        wid = lax.axis_index("s") * _NC + lax.axis_index("c")

        pltpu.sync_copy(iw_hbm.at[pl.ds(wid * b_w, b_w)], idx_v.at[pl.ds(0, b_w)])
        pltpu.sync_copy(cw_hbm.at[pl.ds(wid * b_w, b_w)],
                        idx_v.at[pl.ds(b_w, b_w)])
        pltpu.sync_copy(nz_hbm.at[pl.ds(wid * n_w, n_w)],
                        idx_v.at[pl.ds(2 * b_w, n_w)])

        chunks = [(0, win_hbm, oin_hbm, wid * b_w),
                  (b_w, wctx_hbm, octx_hbm, wid * b_w)]
        for j in range(nneg):
            chunks.append((2 * b_w + j * _CH, wctx_hbm, oneg_hbm,
                           wid * n_w + j * _CH))

        bufs, sems = (rows0, rows1), (sem0, sem1)
        copies = [None] * len(chunks)
        for j, (ioff, table, out, ooff) in enumerate(chunks):
            copies[j] = pltpu.async_copy(
                table.at[idx_v.at[pl.ds(ioff, _CH)]], bufs[j % 2], sems[j % 2])
            if j >= 1:
                _, _, p_out, p_ooff = chunks[j - 1]
                copies[j - 1].wait()
                pltpu.sync_copy(bufs[(j - 1) % 2],
                                p_out.at[pl.ds(p_ooff, _CH), pl.ds(0, _EMB)])
        copies[-1].wait()
        _, _, l_out, l_ooff = chunks[-1]
        pltpu.sync_copy(bufs[(len(chunks) - 1) % 2],
                        l_out.at[pl.ds(l_ooff, _CH), pl.ds(0, _EMB)])

    return k(input_word, context_word, noise_flat, W_in, W_ctx)


def _tc_loss(emb_in, emb_ctx, emb_neg):
    B = _BATCH
    Bb = 1024
    G = B // Bb

    def body(in_ref, ctx_ref, neg_ref, acc_ref):
        a = in_ref[:, : _EMB]
        c = ctx_ref[:, : _EMB]
        n = neg_ref[:, : _EMB].reshape(Bb, _NEG, _EMB)
        pos = jnp.sum(a * c, axis=1, keepdims=True)
        negs = jnp.sum(n * a[:, None, :], axis=2)

        def logsig(x):
            return jnp.minimum(x, 0.0) - jnp.log1p(jnp.exp(-jnp.abs(x)))

        total = jnp.sum(logsig(pos)) + jnp.sum(logsig(-negs))

        @pl.when(pl.program_id(0) == 0)
        def _():
            acc_ref[...] = jnp.zeros((1, 1), jnp.float32)

        acc_ref[...] += jnp.reshape(total, (1, 1))

    acc = pl.pallas_call(
        body,
        grid=(G,),
        in_specs=[
            pl.BlockSpec((Bb, _PAD), lambda i: (i, 0)),
            pl.BlockSpec((Bb, _PAD), lambda i: (i, 0)),
            pl.BlockSpec((Bb * _NEG, _PAD), lambda i: (i, 0)),
        ],
        out_specs=pl.BlockSpec((1, 1), lambda i: (0, 0)),
        out_shape=jax.ShapeDtypeStruct((1, 1), jnp.float32),
    )(emb_in, emb_ctx, emb_neg)
    return -acc[0, 0] / B


def kernel(input_word, context_word, W_in, W_ctx):
    iw = input_word.astype(jnp.int32)
    cw = context_word.astype(jnp.int32)
    nz = _noise_flat()
    emb_in, emb_ctx, emb_neg = _sc_gather(iw, cw, nz, W_in, W_ctx)
    return _tc_loss(emb_in, emb_ctx, emb_neg)
